# baseline (speedup 1.0000x reference)
import functools

import jax
import jax.numpy as jnp
from jax import lax
from jax.experimental import pallas as pl
from jax.experimental.pallas import tpu as pltpu
from jax.experimental.pallas import tpu_sc as plsc

N_CODEBOOKS = 8
VOCAB = 1024
D_MODEL = 1024
B = 16
L = 2048
ROWS = B * L
WORDS = D_MODEL // 2

NUM_CORES = 2
NUM_SUBCORES = 16
NUM_WORKERS = NUM_CORES * NUM_SUBCORES
ROWS_PER_WORKER = ROWS // NUM_WORKERS

CHUNK_ROWS = 8
CHUNK_IDX = CHUNK_ROWS * N_CODEBOOKS
CHUNKS_PER_WORKER = ROWS_PER_WORKER // CHUNK_ROWS
TOTAL_CHUNKS = ROWS // CHUNK_ROWS
LANES = 16
HI_MASK = -65536


def _lo_f32(v):
    return lax.bitcast_convert_type(v << 16, jnp.float32)


def _hi_f32(v):
    return lax.bitcast_convert_type(v & HI_MASK, jnp.float32)


def _make_kernel():
    mesh = plsc.VectorSubcoreMesh(core_axis_name="c", subcore_axis_name="s")

    @functools.partial(
        pl.kernel,
        mesh=mesh,
        out_type=jax.ShapeDtypeStruct((ROWS * D_MODEL,), jnp.float32),
        compiler_params=pltpu.CompilerParams(needs_layout_passes=False),
        scratch_types=[
            pltpu.VMEM((CHUNKS_PER_WORKER, CHUNK_IDX), jnp.int32),
            pltpu.VMEM((CHUNK_IDX, WORDS), jnp.int32),
            pltpu.VMEM((CHUNK_IDX, WORDS), jnp.int32),
            pltpu.VMEM((CHUNK_ROWS * D_MODEL,), jnp.float32),
            pltpu.VMEM((CHUNK_ROWS * D_MODEL,), jnp.float32),
            pltpu.SemaphoreType.DMA,
            pltpu.SemaphoreType.DMA,
            pltpu.SemaphoreType.DMA,
            pltpu.SemaphoreType.DMA,
        ],
    )
    def body(idx_hbm, w_hbm, out_hbm, idx_all, gb0, gb1, ob0, ob1,
             s0, s1, os0, os1):
        gbufs = (gb0, gb1)
        obufs = (ob0, ob1)
        sems = (s0, s1)
        osems = (os0, os1)
        wid = lax.axis_index("s") * NUM_CORES + lax.axis_index("c")
        base_row = wid * ROWS_PER_WORKER
        base_chunk = wid * CHUNKS_PER_WORKER

        pltpu.sync_copy(idx_hbm.at[pl.ds(base_chunk, CHUNKS_PER_WORKER)],
                        idx_all)

        def gather(g, b):
            return pltpu.make_async_copy(
                w_hbm.at[idx_all.at[g]], gbufs[b], sems[b])

        def out_copy(g, b):
            return pltpu.make_async_copy(
                obufs[b],
                out_hbm.at[pl.ds((base_row + g * CHUNK_ROWS) * D_MODEL,
                                 CHUNK_ROWS * D_MODEL)], osems[b])

        gather(0, 0).start()

        def compute(gbuf, obuf):
            def _tree(xs):
                while len(xs) > 1:
                    xs = [xs[i] + xs[i + 1] for i in range(0, len(xs), 2)]
                return xs[0]

            iota2 = lax.iota(jnp.int32, LANES) * 2

            @plsc.parallel_loop(0, WORDS // LANES)
            def col(j):
                pos_e = iota2 + j * 2 * LANES
                pos_o = pos_e + 1
                for c in range(CHUNK_ROWS):
                    r = c * N_CODEBOOKS
                    vs = [gbuf[r + k, pl.ds(j * LANES, LANES)]
                          for k in range(N_CODEBOOKS)]
                    acc_e = _tree([_lo_f32(v) for v in vs])
                    acc_o = _tree([lax.bitcast_convert_type(v, jnp.float32)
                                   for v in vs])
                    plsc.store_scatter(obuf, [pos_e + c * D_MODEL], acc_e)
                    plsc.store_scatter(obuf, [pos_o + c * D_MODEL], acc_o)

        def step(g2, carry):
            for b in range(2):
                g = g2 * 2 + b
                nb = 1 - b

                @pl.when(g + 1 < CHUNKS_PER_WORKER)
                def _():
                    gather(g + 1, nb).start()

                gather(g, b).wait()

                @pl.when(g >= 2)
                def _():
                    out_copy(g - 2, b).wait()

                compute(gbufs[b], obufs[b])
                out_copy(g, b).start()
            return carry

        lax.fori_loop(0, CHUNKS_PER_WORKER // 2, step, 0)
        out_copy(CHUNKS_PER_WORKER - 2, 0).wait()
        out_copy(CHUNKS_PER_WORKER - 1, 1).wait()

    return body


_sc_body = _make_kernel()


def kernel(tokens, W):
    w_bf = W.reshape(N_CODEBOOKS * VOCAB, WORDS, 2).astype(jnp.bfloat16)
    w_i32 = lax.bitcast_convert_type(w_bf, jnp.int32)

    offs = jnp.arange(N_CODEBOOKS, dtype=jnp.int32) * VOCAB
    idx = tokens.transpose(0, 2, 1) + offs[None, None, :]
    idx_chunks = idx.reshape(TOTAL_CHUNKS, CHUNK_IDX)

    out = _sc_body(idx_chunks, w_i32)
    return out.reshape(B, L, D_MODEL)


# speedup vs baseline: 1.3289x; 1.3289x over previous
import functools

import jax
import jax.numpy as jnp
from jax import lax
from jax.experimental import pallas as pl
from jax.experimental.pallas import tpu as pltpu
from jax.experimental.pallas import tpu_sc as plsc

N_CODEBOOKS = 8
VOCAB = 1024
D_MODEL = 1024
B = 16
L = 2048
ROWS = B * L
WORDS = D_MODEL // 2

NUM_CORES = 2
NUM_SUBCORES = 16
NUM_WORKERS = NUM_CORES * NUM_SUBCORES
ROWS_PER_WORKER = ROWS // NUM_WORKERS

CHUNK_ROWS = 8
CHUNK_IDX = CHUNK_ROWS * N_CODEBOOKS
CHUNKS_PER_WORKER = ROWS_PER_WORKER // CHUNK_ROWS
TOTAL_CHUNKS = ROWS // CHUNK_ROWS
LANES = 16
HI_MASK = -65536


def _lo_f32(v):
    return lax.bitcast_convert_type(v << 16, jnp.float32)


def _hi_f32(v):
    return lax.bitcast_convert_type(v & HI_MASK, jnp.float32)


def _make_kernel():
    mesh = plsc.VectorSubcoreMesh(core_axis_name="c", subcore_axis_name="s")

    @functools.partial(
        pl.kernel,
        mesh=mesh,
        out_type=jax.ShapeDtypeStruct((ROWS, D_MODEL), jnp.float32),
        scratch_types=[
            pltpu.VMEM((CHUNKS_PER_WORKER, CHUNK_IDX), jnp.int32),
            pltpu.VMEM((CHUNK_IDX, WORDS), jnp.int32),
            pltpu.VMEM((CHUNK_IDX, WORDS), jnp.int32),
            pltpu.VMEM((CHUNK_ROWS, D_MODEL), jnp.float32),
            pltpu.VMEM((CHUNK_ROWS, D_MODEL), jnp.float32),
            pltpu.SemaphoreType.DMA,
            pltpu.SemaphoreType.DMA,
            pltpu.SemaphoreType.DMA,
            pltpu.SemaphoreType.DMA,
        ],
    )
    def body(idx_hbm, w_hbm, out_hbm, idx_all, gb0, gb1, ob0, ob1,
             s0, s1, os0, os1):
        gbufs = (gb0, gb1)
        obufs = (ob0, ob1)
        sems = (s0, s1)
        osems = (os0, os1)
        wid = lax.axis_index("s") * NUM_CORES + lax.axis_index("c")
        base_row = wid * ROWS_PER_WORKER
        base_chunk = wid * CHUNKS_PER_WORKER

        pltpu.sync_copy(idx_hbm.at[pl.ds(base_chunk, CHUNKS_PER_WORKER)],
                        idx_all)

        def gather(g, b):
            return pltpu.make_async_copy(
                w_hbm.at[idx_all.at[g]], gbufs[b], sems[b])

        def out_copy(g, b):
            return pltpu.make_async_copy(
                obufs[b], out_hbm.at[pl.ds(base_row + g * CHUNK_ROWS,
                                           CHUNK_ROWS)], osems[b])

        gather(0, 0).start()

        def compute(gbuf, obuf):
            def _tree(xs):
                while len(xs) > 1:
                    xs = [xs[i] + xs[i + 1] for i in range(0, len(xs), 2)]
                return xs[0]

            @plsc.parallel_loop(0, WORDS // LANES)
            def col(j):
                for c in range(CHUNK_ROWS):
                    r = c * N_CODEBOOKS
                    vs = [gbuf[r + k, pl.ds(j * LANES, LANES)]
                          for k in range(N_CODEBOOKS)]
                    acc_lo = _tree([_lo_f32(v) for v in vs])
                    acc_hi = _tree([lax.bitcast_convert_type(v, jnp.float32)
                                    for v in vs])
                    obuf[c, pl.ds(j * LANES, LANES)] = acc_lo
                    obuf[c, pl.ds(WORDS + j * LANES, LANES)] = acc_hi

        def step(g2, carry):
            for b in range(2):
                g = g2 * 2 + b
                nb = 1 - b

                @pl.when(g + 1 < CHUNKS_PER_WORKER)
                def _():
                    gather(g + 1, nb).start()

                gather(g, b).wait()

                @pl.when(g >= 2)
                def _():
                    out_copy(g - 2, b).wait()

                compute(gbufs[b], obufs[b])
                out_copy(g, b).start()
            return carry

        lax.fori_loop(0, CHUNKS_PER_WORKER // 2, step, 0)
        out_copy(CHUNKS_PER_WORKER - 2, 0).wait()
        out_copy(CHUNKS_PER_WORKER - 1, 1).wait()

    return body


_sc_body = _make_kernel()


def kernel(tokens, W):
    wu = lax.bitcast_convert_type(
        W.reshape(N_CODEBOOKS * VOCAB, 2, WORDS), jnp.uint32)

    def _rne16(u):
        return (u + jnp.uint32(0x7FFF) + ((u >> 16) & jnp.uint32(1))) >> 16

    shifts = jnp.array([1, 65536], dtype=jnp.uint32)[None, :, None]
    w_i32 = lax.bitcast_convert_type(
        jnp.sum(_rne16(wu) * shifts, axis=1, dtype=jnp.uint32), jnp.int32)

    offs = jnp.arange(N_CODEBOOKS, dtype=jnp.int32) * VOCAB
    idx = tokens.transpose(0, 2, 1) + offs[None, None, :]
    idx_chunks = idx.reshape(TOTAL_CHUNKS, CHUNK_IDX)

    out = _sc_body(idx_chunks, w_i32)
    return out.reshape(B, L, D_MODEL)


# speedup vs baseline: 1.3892x; 1.0454x over previous
import functools

import jax
import jax.numpy as jnp
from jax import lax
from jax.experimental import pallas as pl
from jax.experimental.pallas import tpu as pltpu
from jax.experimental.pallas import tpu_sc as plsc

N_CODEBOOKS = 8
VOCAB = 1024
D_MODEL = 1024
B = 16
L = 2048
ROWS = B * L
WORDS = D_MODEL // 2

NUM_CORES = 2
NUM_SUBCORES = 16
NUM_WORKERS = NUM_CORES * NUM_SUBCORES
ROWS_PER_WORKER = ROWS // NUM_WORKERS

CHUNK_ROWS = 8
CHUNK_IDX = CHUNK_ROWS * N_CODEBOOKS
CHUNKS_PER_WORKER = ROWS_PER_WORKER // CHUNK_ROWS
TOTAL_CHUNKS = ROWS // CHUNK_ROWS
LANES = 16
HI_MASK = -65536


def _lo_f32(v):
    return lax.bitcast_convert_type(v << 16, jnp.float32)


def _hi_f32(v):
    return lax.bitcast_convert_type(v & HI_MASK, jnp.float32)


def _make_kernel():
    mesh = plsc.VectorSubcoreMesh(core_axis_name="c", subcore_axis_name="s")

    @functools.partial(
        pl.kernel,
        mesh=mesh,
        out_type=jax.ShapeDtypeStruct((ROWS, D_MODEL), jnp.float32),
        scratch_types=[
            pltpu.VMEM((CHUNKS_PER_WORKER, CHUNK_IDX), jnp.int32),
            pltpu.VMEM((CHUNK_IDX, WORDS), jnp.int32),
            pltpu.VMEM((CHUNK_IDX, WORDS), jnp.int32),
            pltpu.VMEM((CHUNK_ROWS, D_MODEL), jnp.float32),
            pltpu.VMEM((CHUNK_ROWS, D_MODEL), jnp.float32),
            pltpu.SemaphoreType.DMA,
            pltpu.SemaphoreType.DMA,
            pltpu.SemaphoreType.DMA,
            pltpu.SemaphoreType.DMA,
        ],
    )
    def body(idx_hbm, w_hbm, out_hbm, idx_all, gb0, gb1, ob0, ob1,
             s0, s1, os0, os1):
        gbufs = (gb0, gb1)
        obufs = (ob0, ob1)
        sems = (s0, s1)
        osems = (os0, os1)
        wid = lax.axis_index("s") * NUM_CORES + lax.axis_index("c")
        base_row = wid * ROWS_PER_WORKER
        base_chunk = wid * CHUNKS_PER_WORKER

        pltpu.sync_copy(idx_hbm.at[pl.ds(base_chunk, CHUNKS_PER_WORKER)],
                        idx_all)

        def gather(g, b):
            return pltpu.make_async_copy(
                w_hbm.at[idx_all.at[g]], gbufs[b], sems[b])

        def out_copy(g, b):
            return pltpu.make_async_copy(
                obufs[b], out_hbm.at[pl.ds(base_row + g * CHUNK_ROWS,
                                           CHUNK_ROWS)], osems[b])

        gather(0, 0).start()

        def compute(gbuf, obuf):
            def _tree(xs):
                while len(xs) > 1:
                    xs = [xs[i] + xs[i + 1] for i in range(0, len(xs), 2)]
                return xs[0]

            @plsc.parallel_loop(0, WORDS // LANES)
            def col(j):
                for c in range(CHUNK_ROWS):
                    r = c * N_CODEBOOKS
                    vs = [gbuf[r + k, pl.ds(j * LANES, LANES)]
                          for k in range(N_CODEBOOKS)]
                    acc_lo = _tree([_lo_f32(v) for v in vs])
                    acc_hi = _tree([lax.bitcast_convert_type(v, jnp.float32)
                                    for v in vs])
                    obuf[c, pl.ds(j * LANES, LANES)] = acc_lo
                    obuf[c, pl.ds(WORDS + j * LANES, LANES)] = acc_hi

        def step(g2, carry):
            for b in range(2):
                g = g2 * 2 + b
                nb = 1 - b

                @pl.when(g + 1 < CHUNKS_PER_WORKER)
                def _():
                    gather(g + 1, nb).start()

                gather(g, b).wait()

                @pl.when(g >= 2)
                def _():
                    out_copy(g - 2, b).wait()

                compute(gbufs[b], obufs[b])
                out_copy(g, b).start()
            return carry

        lax.fori_loop(0, CHUNKS_PER_WORKER // 2, step, 0)
        out_copy(CHUNKS_PER_WORKER - 2, 0).wait()
        out_copy(CHUNKS_PER_WORKER - 1, 1).wait()

    return body


_sc_body = _make_kernel()


def kernel(tokens, W):
    wu = lax.bitcast_convert_type(
        W.reshape(N_CODEBOOKS * VOCAB, 2, WORDS), jnp.uint32)

    def _rne16(u):
        return (u + jnp.uint32(0x7FFF) + ((u >> 16) & jnp.uint32(1))) >> 16

    w_i32 = lax.bitcast_convert_type(
        _rne16(wu[:, 0, :]) | (_rne16(wu[:, 1, :]) << 16), jnp.int32)

    offs = jnp.arange(N_CODEBOOKS, dtype=jnp.int32) * VOCAB
    idx = tokens.transpose(0, 2, 1) + offs[None, None, :]
    idx_chunks = idx.reshape(TOTAL_CHUNKS, CHUNK_IDX)

    out = _sc_body(idx_chunks, w_i32)
    return out.reshape(B, L, D_MODEL)


# speedup vs baseline: 1.5512x; 1.1166x over previous
import functools

import jax
import jax.numpy as jnp
from jax import lax
from jax.experimental import pallas as pl
from jax.experimental.pallas import tpu as pltpu
from jax.experimental.pallas import tpu_sc as plsc

N_CODEBOOKS = 8
VOCAB = 1024
D_MODEL = 1024
B = 16
L = 2048
ROWS = B * L
WORDS = D_MODEL // 2

NUM_CORES = 2
NUM_SUBCORES = 16
NUM_WORKERS = NUM_CORES * NUM_SUBCORES
ROWS_PER_WORKER = ROWS // NUM_WORKERS

CHUNK_ROWS = 8
CHUNK_IDX = CHUNK_ROWS * N_CODEBOOKS
CHUNKS_PER_WORKER = ROWS_PER_WORKER // CHUNK_ROWS
TOTAL_CHUNKS = ROWS // CHUNK_ROWS
LANES = 16
HI_MASK = -65536

PACK_ROWS = 8
PACK_ITERS = N_CODEBOOKS * VOCAB // NUM_SUBCORES // PACK_ROWS


def _lo_f32(v):
    return lax.bitcast_convert_type(v << 16, jnp.float32)


def _hi_f32(v):
    return lax.bitcast_convert_type(v & HI_MASK, jnp.float32)


def _make_kernel():
    mesh = plsc.VectorSubcoreMesh(core_axis_name="c", subcore_axis_name="s")

    @functools.partial(
        pl.kernel,
        mesh=mesh,
        out_type=(jax.ShapeDtypeStruct((ROWS, D_MODEL), jnp.float32),
                  jax.ShapeDtypeStruct((N_CODEBOOKS * VOCAB, WORDS),
                                       jnp.int32)),
        scratch_types=[
            pltpu.VMEM((CHUNKS_PER_WORKER, CHUNK_IDX), jnp.int32),
            pltpu.VMEM((CHUNK_IDX, WORDS), jnp.int32),
            pltpu.VMEM((CHUNK_IDX, WORDS), jnp.int32),
            pltpu.VMEM((CHUNK_ROWS, D_MODEL), jnp.float32),
            pltpu.VMEM((CHUNK_ROWS, D_MODEL), jnp.float32),
            pltpu.VMEM((PACK_ROWS, D_MODEL), jnp.int32),
            pltpu.VMEM((PACK_ROWS, D_MODEL), jnp.int32),
            pltpu.VMEM((PACK_ROWS, WORDS), jnp.int32),
            pltpu.VMEM((PACK_ROWS, WORDS), jnp.int32),
            pltpu.SemaphoreType.DMA,
            pltpu.SemaphoreType.DMA,
            pltpu.SemaphoreType.DMA,
            pltpu.SemaphoreType.DMA,
        ],
    )
    def body(idx_hbm, wraw_hbm, out_hbm, wpk_hbm, idx_all, gb0, gb1,
             ob0, ob1, pin0, pin1, pout0, pout1, s0, s1, os0, os1):
        gbufs = (gb0, gb1)
        obufs = (ob0, ob1)
        pins = (pin0, pin1)
        pouts = (pout0, pout1)
        sems = (s0, s1)
        osems = (os0, os1)
        sid = lax.axis_index("s")
        wid = sid * NUM_CORES + lax.axis_index("c")
        base_row = wid * ROWS_PER_WORKER
        base_chunk = wid * CHUNKS_PER_WORKER

        def _rne16(u):
            rb = lax.shift_right_logical(u, 16) & 1
            return lax.shift_right_logical(u + 32767 + rb, 16)

        prow = sid * (N_CODEBOOKS * VOCAB // NUM_SUBCORES)

        def pin_copy(i, b):
            return pltpu.make_async_copy(
                wraw_hbm.at[pl.ds(prow + i * PACK_ROWS, PACK_ROWS)],
                pins[b], sems[b])

        def pout_copy(i, b):
            return pltpu.make_async_copy(
                pouts[b],
                wpk_hbm.at[pl.ds(prow + i * PACK_ROWS, PACK_ROWS)],
                osems[b])

        def pack_compute(pin, pout):
            @plsc.parallel_loop(0, D_MODEL // 32)
            def pcol(g):
                for c in range(PACK_ROWS):
                    lo = _rne16(pin[c, pl.ds(g * 32, LANES)])
                    hi = _rne16(pin[c, pl.ds(g * 32 + LANES, LANES)])
                    pout[c, pl.ds(g * LANES, LANES)] = lo | (hi << 16)

        pin_copy(0, 0).start()

        def pstep(i2, carry):
            for b in range(2):
                i = i2 * 2 + b

                @pl.when(i + 1 < PACK_ITERS)
                def _():
                    pin_copy(i + 1, 1 - b).start()

                pin_copy(i, b).wait()

                @pl.when(i >= 2)
                def _():
                    pout_copy(i - 2, b).wait()

                pack_compute(pins[b], pouts[b])
                pout_copy(i, b).start()
            return carry

        lax.fori_loop(0, PACK_ITERS // 2, pstep, 0)
        pout_copy(PACK_ITERS - 2, 0).wait()
        pout_copy(PACK_ITERS - 1, 1).wait()
        plsc.subcore_barrier()
        w_hbm = wpk_hbm

        pltpu.sync_copy(idx_hbm.at[pl.ds(base_chunk, CHUNKS_PER_WORKER)],
                        idx_all)

        def gather(g, b):
            return pltpu.make_async_copy(
                w_hbm.at[idx_all.at[g]], gbufs[b], sems[b])

        def out_copy(g, b):
            return pltpu.make_async_copy(
                obufs[b], out_hbm.at[pl.ds(base_row + g * CHUNK_ROWS,
                                           CHUNK_ROWS)], osems[b])

        gather(0, 0).start()

        def compute(gbuf, obuf):
            def _tree(xs):
                while len(xs) > 1:
                    xs = [xs[i] + xs[i + 1] for i in range(0, len(xs), 2)]
                return xs[0]

            @plsc.parallel_loop(0, WORDS // LANES)
            def col(j):
                for c in range(CHUNK_ROWS):
                    r = c * N_CODEBOOKS
                    vs = [gbuf[r + k, pl.ds(j * LANES, LANES)]
                          for k in range(N_CODEBOOKS)]
                    acc_e = _tree([_lo_f32(v) for v in vs])
                    acc_o = _tree([lax.bitcast_convert_type(v, jnp.float32)
                                   for v in vs])
                    obuf[c, pl.ds(j * 2 * LANES, LANES)] = acc_e
                    obuf[c, pl.ds(j * 2 * LANES + LANES, LANES)] = acc_o

        def step(g2, carry):
            for b in range(2):
                g = g2 * 2 + b
                nb = 1 - b

                @pl.when(g + 1 < CHUNKS_PER_WORKER)
                def _():
                    gather(g + 1, nb).start()

                gather(g, b).wait()

                @pl.when(g >= 2)
                def _():
                    out_copy(g - 2, b).wait()

                compute(gbufs[b], obufs[b])
                out_copy(g, b).start()
            return carry

        lax.fori_loop(0, CHUNKS_PER_WORKER // 2, step, 0)
        out_copy(CHUNKS_PER_WORKER - 2, 0).wait()
        out_copy(CHUNKS_PER_WORKER - 1, 1).wait()

    return body


_sc_body = _make_kernel()


def kernel(tokens, W):
    w_raw = lax.bitcast_convert_type(
        W.reshape(N_CODEBOOKS * VOCAB, D_MODEL), jnp.int32)

    offs = jnp.arange(N_CODEBOOKS, dtype=jnp.int32) * VOCAB
    idx = tokens.transpose(0, 2, 1) + offs[None, None, :]
    idx_chunks = idx.reshape(TOTAL_CHUNKS, CHUNK_IDX)

    out, _ = _sc_body(idx_chunks, w_raw)
    return out.reshape(B, L, D_MODEL)


# speedup vs baseline: 1.5608x; 1.0061x over previous
import functools

import jax
import jax.numpy as jnp
from jax import lax
from jax.experimental import pallas as pl
from jax.experimental.pallas import tpu as pltpu
from jax.experimental.pallas import tpu_sc as plsc

N_CODEBOOKS = 8
VOCAB = 1024
D_MODEL = 1024
B = 16
L = 2048
ROWS = B * L
WORDS = D_MODEL // 2

NUM_CORES = 2
NUM_SUBCORES = 16
NUM_WORKERS = NUM_CORES * NUM_SUBCORES
ROWS_PER_WORKER = ROWS // NUM_WORKERS

CHUNK_ROWS = 8
CHUNK_IDX = CHUNK_ROWS * N_CODEBOOKS
CHUNKS_PER_WORKER = ROWS_PER_WORKER // CHUNK_ROWS
TOTAL_CHUNKS = ROWS // CHUNK_ROWS
LANES = 16

PACK_ROWS = 8
PACK_ITERS = N_CODEBOOKS * VOCAB // NUM_SUBCORES // PACK_ROWS


def _lo_f32(v):
    return lax.bitcast_convert_type(v << 16, jnp.float32)


def _make_kernel():
    mesh = plsc.VectorSubcoreMesh(core_axis_name="c", subcore_axis_name="s")

    @functools.partial(
        pl.kernel,
        mesh=mesh,
        out_type=(jax.ShapeDtypeStruct((ROWS, D_MODEL), jnp.float32),
                  jax.ShapeDtypeStruct((N_CODEBOOKS * VOCAB, WORDS),
                                       jnp.int32)),
        scratch_types=[
            pltpu.VMEM((CHUNKS_PER_WORKER, CHUNK_IDX), jnp.int32),
            pltpu.VMEM((CHUNK_IDX, WORDS), jnp.int32),
            pltpu.VMEM((CHUNK_IDX, WORDS), jnp.int32),
            pltpu.VMEM((CHUNK_ROWS, D_MODEL), jnp.float32),
            pltpu.VMEM((CHUNK_ROWS, D_MODEL), jnp.float32),
            pltpu.VMEM((PACK_ROWS, D_MODEL), jnp.int32),
            pltpu.VMEM((PACK_ROWS, D_MODEL), jnp.int32),
            pltpu.VMEM((PACK_ROWS, WORDS), jnp.int32),
            pltpu.VMEM((PACK_ROWS, WORDS), jnp.int32),
            pltpu.SemaphoreType.DMA,
            pltpu.SemaphoreType.DMA,
            pltpu.SemaphoreType.DMA,
            pltpu.SemaphoreType.DMA,
        ],
    )
    def body(idx_hbm, wraw_hbm, out_hbm, wpk_hbm, idx_all, gb0, gb1,
             ob0, ob1, pin0, pin1, pout0, pout1, s0, s1, os0, os1):
        gbufs = (gb0, gb1)
        obufs = (ob0, ob1)
        pins = (pin0, pin1)
        pouts = (pout0, pout1)
        sems = (s0, s1)
        osems = (os0, os1)
        sid = lax.axis_index("s")
        wid = sid * NUM_CORES + lax.axis_index("c")
        base_row = wid * ROWS_PER_WORKER
        base_chunk = wid * CHUNKS_PER_WORKER

        def _rne16(u):
            rb = lax.shift_right_logical(u, 16) & 1
            return lax.shift_right_logical(u + 32767 + rb, 16)

        prow = sid * (N_CODEBOOKS * VOCAB // NUM_SUBCORES)

        def pin_copy(i, b):
            return pltpu.make_async_copy(
                wraw_hbm.at[pl.ds(prow + i * PACK_ROWS, PACK_ROWS)],
                pins[b], sems[b])

        def pout_copy(i, b):
            return pltpu.make_async_copy(
                pouts[b],
                wpk_hbm.at[pl.ds(prow + i * PACK_ROWS, PACK_ROWS)],
                osems[b])

        def pack_compute(pin, pout):
            @plsc.parallel_loop(0, D_MODEL // 32)
            def pcol(g):
                for c in range(PACK_ROWS):
                    lo = _rne16(pin[c, pl.ds(g * 32, LANES)])
                    hi = _rne16(pin[c, pl.ds(g * 32 + LANES, LANES)])
                    pout[c, pl.ds(g * LANES, LANES)] = lo | (hi << 16)

        pin_copy(0, 0).start()

        def pstep(i2, carry):
            for b in range(2):
                i = i2 * 2 + b

                @pl.when(i + 1 < PACK_ITERS)
                def _():
                    pin_copy(i + 1, 1 - b).start()

                pin_copy(i, b).wait()

                @pl.when(i >= 2)
                def _():
                    pout_copy(i - 2, b).wait()

                pack_compute(pins[b], pouts[b])
                pout_copy(i, b).start()
            return carry

        lax.fori_loop(0, PACK_ITERS // 2, pstep, 0)
        pout_copy(PACK_ITERS - 2, 0).wait()
        pout_copy(PACK_ITERS - 1, 1).wait()
        plsc.subcore_barrier()
        w_hbm = wpk_hbm

        pltpu.sync_copy(idx_hbm.at[pl.ds(base_chunk, CHUNKS_PER_WORKER)],
                        idx_all)

        def gather(g, b):
            return pltpu.make_async_copy(
                w_hbm.at[idx_all.at[g]], gbufs[b], sems[b])

        def out_copy(g, b):
            return pltpu.make_async_copy(
                obufs[b], out_hbm.at[pl.ds(base_row + g * CHUNK_ROWS,
                                           CHUNK_ROWS)], osems[b])

        gather(0, 0).start()

        def compute(gbuf, obuf):
            def _tree(xs):
                while len(xs) > 1:
                    xs = [xs[i] + xs[i + 1] for i in range(0, len(xs), 2)]
                return xs[0]

            @plsc.parallel_loop(0, WORDS // LANES)
            def col(j):
                for c in range(CHUNK_ROWS):
                    r = c * N_CODEBOOKS
                    vs = [gbuf[r + k, pl.ds(j * LANES, LANES)]
                          for k in range(N_CODEBOOKS)]
                    acc_e = _tree([_lo_f32(v) for v in vs])
                    acc_o = _tree([lax.bitcast_convert_type(v, jnp.float32)
                                   for v in vs])
                    obuf[c, pl.ds(j * 2 * LANES, LANES)] = acc_e
                    obuf[c, pl.ds(j * 2 * LANES + LANES, LANES)] = acc_o

        def step(g2, carry):
            for b in range(2):
                g = g2 * 2 + b
                nb = 1 - b

                @pl.when(g + 1 < CHUNKS_PER_WORKER)
                def _():
                    gather(g + 1, nb).start()

                gather(g, b).wait()

                @pl.when(g >= 2)
                def _():
                    out_copy(g - 2, b).wait()

                compute(gbufs[b], obufs[b])
                out_copy(g, b).start()
            return carry

        lax.fori_loop(0, CHUNKS_PER_WORKER // 2, step, 0)
        out_copy(CHUNKS_PER_WORKER - 2, 0).wait()
        out_copy(CHUNKS_PER_WORKER - 1, 1).wait()

    return body


_sc_body = _make_kernel()


def kernel(tokens, W):
    w_raw = lax.bitcast_convert_type(
        W.reshape(N_CODEBOOKS * VOCAB, D_MODEL), jnp.int32)

    offs = jnp.arange(N_CODEBOOKS, dtype=jnp.int32) * VOCAB
    idx = tokens.transpose(0, 2, 1) + offs[None, None, :]
    idx_chunks = idx.reshape(TOTAL_CHUNKS, CHUNK_IDX)

    out, _ = _sc_body(idx_chunks, w_raw)
    return out.reshape(B, L, D_MODEL)
